# baseline (speedup 1.0000x reference)
import functools
import math

import jax
import jax.numpy as jnp
from jax import lax
from jax.experimental import pallas as pl
from jax.experimental.pallas import tpu as pltpu
from jax.experimental.pallas import tpu_sc as plsc

VOCAB = 100000
D_MODEL = 1024
BATCH = 4
SEQ_LEN = 4096
N_ROWS = BATCH * SEQ_LEN
SCALE = math.sqrt(D_MODEL)

_info = plsc.get_sparse_core_info()
NUM_CORES = _info.num_cores
NUM_SUBCORES = _info.num_subcores
LANES = _info.num_lanes
NW = NUM_CORES * NUM_SUBCORES
ROWS_PER_W = N_ROWS // NW
K = 32
N_CHUNKS = ROWS_PER_W // K
VECS_PER_ROW = D_MODEL // LANES


def _emb_body(x_ref, tok_ref, pos_ref, out_ref, idx_v, rows_v, pos_v, sem):
    wid = lax.axis_index("s") * NUM_CORES + lax.axis_index("c")
    base = wid * ROWS_PER_W
    s_base = base % SEQ_LEN

    def chunk_body(c, carry):
        row0 = base + c * K
        p0 = s_base + c * K
        pltpu.sync_copy(x_ref.at[pl.ds(row0, K)], idx_v)
        gather = pltpu.async_copy(tok_ref.at[idx_v], rows_v, sem)
        pltpu.sync_copy(pos_ref.at[pl.ds(p0, K)], pos_v)
        gather.wait()

        def row_body(r, carry2):
            for v in range(VECS_PER_ROW):
                sl = pl.ds(v * LANES, LANES)
                rows_v[r, sl] = rows_v[r, sl] * SCALE + pos_v[r, sl]
            return carry2

        lax.fori_loop(0, K, row_body, 0, unroll=1)
        pltpu.sync_copy(rows_v, out_ref.at[pl.ds(row0, K)])
        return carry

    lax.fori_loop(0, N_CHUNKS, chunk_body, 0, unroll=1)


@jax.jit
def _emb_call(x_flat, token_table, pos_table):
    mesh = plsc.VectorSubcoreMesh(core_axis_name="c", subcore_axis_name="s")
    f = functools.partial(
        pl.kernel,
        out_type=jax.ShapeDtypeStruct((N_ROWS, D_MODEL), jnp.float32),
        mesh=mesh,
        scratch_types=[
            pltpu.VMEM((K,), jnp.int32),
            pltpu.VMEM((K, D_MODEL), jnp.float32),
            pltpu.VMEM((K, D_MODEL), jnp.float32),
            pltpu.SemaphoreType.DMA,
        ],
    )(_emb_body)
    return f(x_flat, token_table, pos_table)


def kernel(x, token_table, pos_table):
    x_flat = x.reshape(N_ROWS).astype(jnp.int32)
    out = _emb_call(x_flat, token_table, pos_table)
    return out.reshape(BATCH, SEQ_LEN, D_MODEL)


# speedup vs baseline: 1.5446x; 1.5446x over previous
import functools
import math

import jax
import jax.numpy as jnp
from jax import lax
from jax.experimental import pallas as pl
from jax.experimental.pallas import tpu as pltpu
from jax.experimental.pallas import tpu_sc as plsc

VOCAB = 100000
D_MODEL = 1024
BATCH = 4
SEQ_LEN = 4096
N_ROWS = BATCH * SEQ_LEN
SCALE = math.sqrt(D_MODEL)

_info = plsc.get_sparse_core_info()
NUM_CORES = _info.num_cores
NUM_SUBCORES = _info.num_subcores
LANES = _info.num_lanes
NW = NUM_CORES * NUM_SUBCORES
ROWS_PER_W = N_ROWS // NW
K = 16
N_UNITS = ROWS_PER_W // K
VECS_PER_ROW = D_MODEL // LANES


def _emb_body(x_ref, tok_ref, pos_ref, out_ref,
              idx2d, rows0, rows1, pos0, pos1,
              semg0, semg1, semp0, semp1, semw0, semw1):
    wid = lax.axis_index("s") * NUM_CORES + lax.axis_index("c")
    base = wid * ROWS_PER_W
    pltpu.sync_copy(x_ref.at[pl.ds(wid * N_UNITS, N_UNITS)], idx2d)

    bufs = ((rows0, pos0, semg0, semp0, semw0),
            (rows1, pos1, semg1, semp1, semw1))

    def gather_desc(u, b):
        rows_b, _, semg_b, _, _ = bufs[b]
        return pltpu.make_async_copy(tok_ref.at[idx2d.at[u]], rows_b, semg_b)

    def pos_desc(u, b):
        _, pos_b, _, semp_b, _ = bufs[b]
        p0 = lax.rem(base + u * K, SEQ_LEN)
        return pltpu.make_async_copy(pos_ref.at[pl.ds(p0, K)], pos_b, semp_b)

    def wb_desc(u, b):
        rows_b, _, _, _, semw_b = bufs[b]
        row0 = base + u * K
        return pltpu.make_async_copy(rows_b, out_ref.at[pl.ds(row0, K)], semw_b)

    def compute(b):
        rows_b, pos_b = bufs[b][0], bufs[b][1]

        def row_body(r, carry):
            for v in range(VECS_PER_ROW):
                sl = pl.ds(v * LANES, LANES)
                rows_b[r, sl] = rows_b[r, sl] * SCALE + pos_b[r, sl]
            return carry

        lax.fori_loop(0, K, row_body, 0, unroll=1)

    def pair_body(p, carry):
        for b in (0, 1):
            u = 2 * p + b
            @pl.when(p >= 1)
            def _():
                wb_desc(u - 2, b).wait()
            gather_desc(u, b).start()
            pos_desc(u, b).start()
            ob = 1 - b
            @pl.when((p >= 1) | (b == 1))
            def _():
                up = u - 1
                gather_desc(up, ob).wait()
                pos_desc(up, ob).wait()
                compute(ob)
                wb_desc(up, ob).start()
        return carry

    lax.fori_loop(0, N_UNITS // 2, pair_body, 0, unroll=1)

    last = N_UNITS - 1
    gather_desc(last, 1).wait()
    pos_desc(last, 1).wait()
    compute(1)
    wb_desc(last, 1).start()
    wb_desc(last - 1, 0).wait()
    wb_desc(last, 1).wait()


@jax.jit
def _emb_call(x2d, token_table, pos_table):
    mesh = plsc.VectorSubcoreMesh(core_axis_name="c", subcore_axis_name="s")
    f = functools.partial(
        pl.kernel,
        out_type=jax.ShapeDtypeStruct((N_ROWS, D_MODEL), jnp.float32),
        mesh=mesh,
        scratch_types=[
            pltpu.VMEM((N_UNITS, K), jnp.int32),
            pltpu.VMEM((K, D_MODEL), jnp.float32),
            pltpu.VMEM((K, D_MODEL), jnp.float32),
            pltpu.VMEM((K, D_MODEL), jnp.float32),
            pltpu.VMEM((K, D_MODEL), jnp.float32),
            pltpu.SemaphoreType.DMA,
            pltpu.SemaphoreType.DMA,
            pltpu.SemaphoreType.DMA,
            pltpu.SemaphoreType.DMA,
            pltpu.SemaphoreType.DMA,
            pltpu.SemaphoreType.DMA,
        ],
    )(_emb_body)
    return f(x2d, token_table, pos_table)


def kernel(x, token_table, pos_table):
    x2d = x.reshape(N_ROWS // K, K).astype(jnp.int32)
    out = _emb_call(x2d, token_table, pos_table)
    return out.reshape(BATCH, SEQ_LEN, D_MODEL)
